# baseline (speedup 1.0000x reference)
import functools
import math

import jax
import jax.numpy as jnp
from jax import lax
from jax.experimental import pallas as pl
from jax.experimental.pallas import tpu as pltpu
from jax.experimental.pallas import tpu_sc as plsc

_NUM_LAYERS = 4
_ALPHA = 0.1
_THETA = 0.5

_NC = 2
_NS = 16
_L = 16

_K = 80
_NBUF = 5


def _make_sc_prop(N, Np, E, F):
    F2 = F // 2
    eps = E // _NS
    ch = eps // _K
    rps = Np // _NS
    zr = 40
    assert eps * _NS == E and ch * _K == eps and rps * _NS == Np
    assert rps % zr == 0 and rps % 8 == 0 and F2 % _L == 0
    assert ch % _NBUF == 0
    mesh = plsc.VectorSubcoreMesh(core_axis_name="c", subcore_axis_name="s")

    @functools.partial(
        pl.kernel,
        out_type=(jax.ShapeDtypeStruct((Np, F2), jnp.float32),
                  jax.ShapeDtypeStruct((Np, F2), jnp.float32)),
        mesh=mesh,
        compiler_params=pltpu.CompilerParams(
            needs_layout_passes=False, use_tc_tiling_on_sc=False),
        scratch_types=(
            [pltpu.VMEM((ch, _K), jnp.int32),
             pltpu.VMEM((ch, _K), jnp.int32),
             pltpu.VMEM((eps,), jnp.float32)]
            + [pltpu.VMEM((_K, F2), jnp.float32) for _ in range(_NBUF)]
            + [pltpu.VMEM((zr, F2), jnp.float32),
               pltpu.VMEM_SHARED((Np, F2), jnp.float32),
               pltpu.SemaphoreType.DMA((_NBUF,)),
               pltpu.SemaphoreType.DMA((_NBUF,))]
        ),
    )
    def sc_prop(hlo_hbm, hhi_hbm, src_hbm, dst_hbm, ew_hbm, plo_hbm, phi_hbm,
                src_v, dst_v, ew_v, *bufs):
        rows = list(bufs[:_NBUF])
        zero_v, acc_sh, gsem, ssem = bufs[_NBUF:]
        cid = lax.axis_index("c")
        sid = lax.axis_index("s")

        pltpu.sync_copy(src_hbm.at[sid], src_v)
        pltpu.sync_copy(dst_hbm.at[sid], dst_v)
        pltpu.sync_copy(ew_hbm.at[sid], ew_v)

        @pl.loop(0, zr)
        def _(r):
            for j in range(F2 // _L):
                zero_v[r, pl.ds(j * _L, _L)] = jnp.zeros((_L,), jnp.float32)

        @pl.loop(0, rps // zr)
        def _(i):
            pltpu.sync_copy(zero_v, acc_sh.at[pl.ds(sid * rps + i * zr, zr)])

        plsc.subcore_barrier()

        def scale_rows(rows_ref, t):
            @pl.loop(0, _K // _L)
            def _(q):
                wv = ew_v[pl.ds(t * _K + q * _L, _L)]
                for i in range(_L):
                    w = lax.broadcast_in_dim(wv[i], (_L,), ())
                    for j in range(F2 // _L):
                        rows_ref[q * _L + i, pl.ds(j * _L, _L)] = (
                            rows_ref[q * _L + i, pl.ds(j * _L, _L)] * w)

        def main_pipeline(h_hbm):
            def g_fire(b, t):
                pltpu.async_copy(h_hbm.at[src_v.at[t]], rows[b], gsem.at[b])

            def g_wait(b, t):
                pltpu.make_async_copy(
                    h_hbm.at[src_v.at[t]], rows[b], gsem.at[b]).wait()

            def s_fire(b, t):
                pltpu.async_copy(rows[b], acc_sh.at[dst_v.at[t]],
                                 ssem.at[b], add=True)

            def s_wait(b, t):
                pltpu.make_async_copy(rows[b], acc_sh.at[dst_v.at[t]],
                                      ssem.at[b]).wait()

            for b in range(_NBUF):
                g_fire(b, b)

            @pl.loop(0, ch, step=_NBUF)
            def _(t):
                for b in range(_NBUF):
                    cur = t + b
                    g_wait(b, cur)
                    scale_rows(rows[b], cur)
                    s_fire(b, cur)
                    if b == 0:
                        @pl.when(t > 0)
                        def _():
                            s_wait(_NBUF - 1, t - 1)
                            g_fire(_NBUF - 1,
                                   jnp.minimum(t - 1 + _NBUF, ch - 1))
                    else:
                        s_wait(b - 1, cur - 1)
                        g_fire(b - 1, jnp.minimum(cur - 1 + _NBUF, ch - 1))

            s_wait(_NBUF - 1, ch - 1)
            for b in range(_NBUF - 1):
                g_wait(b, ch - 1)

        @pl.when(cid == 0)
        def _():
            main_pipeline(hlo_hbm)

        @pl.when(cid == 1)
        def _():
            main_pipeline(hhi_hbm)

        plsc.subcore_barrier()

        @pl.when(cid == 0)
        def _():
            pltpu.sync_copy(acc_sh.at[pl.ds(sid * rps, rps)],
                            plo_hbm.at[pl.ds(sid * rps, rps)])

        @pl.when(cid == 1)
        def _():
            pltpu.sync_copy(acc_sh.at[pl.ds(sid * rps, rps)],
                            phi_hbm.at[pl.ds(sid * rps, rps)])

    return sc_prop


def _make_lin0(N, F, H, bn):
    H2 = H // 2

    def body(x_ref, w_ref, b_ref, lo_ref, hi_ref):
        h = jnp.maximum(
            jnp.dot(x_ref[...], w_ref[...], preferred_element_type=jnp.float32)
            + b_ref[...], 0.0)
        lo_ref[...] = h[:, :H2]
        hi_ref[...] = h[:, H2:]

    return pl.pallas_call(
        body,
        grid=(N // bn,),
        in_specs=[pl.BlockSpec((bn, F), lambda i: (i, 0)),
                  pl.BlockSpec((F, H), lambda i: (0, 0)),
                  pl.BlockSpec((1, H), lambda i: (0, 0))],
        out_specs=[pl.BlockSpec((bn, H2), lambda i: (i, 0)),
                   pl.BlockSpec((bn, H2), lambda i: (i, 0))],
        out_shape=[jax.ShapeDtypeStruct((N, H2), jnp.float32),
                   jax.ShapeDtypeStruct((N, H2), jnp.float32)],
    )


def _make_layer(N, Np, H, bn, beta):
    H2 = H // 2

    def body(plo_ref, phi_ref, xlo_ref, xhi_ref, w_ref, lo_ref, hi_ref):
        agg = jnp.concatenate([plo_ref[...], phi_ref[...]], axis=1)
        x0 = jnp.concatenate([xlo_ref[...], xhi_ref[...]], axis=1)
        hh = (1.0 - _ALPHA) * agg + _ALPHA * x0
        out = (1.0 - beta) * hh + beta * jnp.dot(
            hh, w_ref[...], preferred_element_type=jnp.float32)
        out = jnp.maximum(out, 0.0)
        lo_ref[...] = out[:, :H2]
        hi_ref[...] = out[:, H2:]

    return pl.pallas_call(
        body,
        grid=(N // bn,),
        in_specs=[pl.BlockSpec((bn, H2), lambda i: (i, 0)),
                  pl.BlockSpec((bn, H2), lambda i: (i, 0)),
                  pl.BlockSpec((bn, H2), lambda i: (i, 0)),
                  pl.BlockSpec((bn, H2), lambda i: (i, 0)),
                  pl.BlockSpec((H, H), lambda i: (0, 0))],
        out_specs=[pl.BlockSpec((bn, H2), lambda i: (i, 0)),
                   pl.BlockSpec((bn, H2), lambda i: (i, 0))],
        out_shape=[jax.ShapeDtypeStruct((N, H2), jnp.float32),
                   jax.ShapeDtypeStruct((N, H2), jnp.float32)],
    )


def _make_final(N, H, C, bn):
    H2 = H // 2

    def body(lo_ref, hi_ref, w_ref, b_ref, o_ref):
        h = jnp.concatenate([lo_ref[...], hi_ref[...]], axis=1)
        logits = jnp.dot(h, w_ref[...],
                         preferred_element_type=jnp.float32) + b_ref[...]
        z = logits - jnp.max(logits, axis=-1, keepdims=True)
        o_ref[...] = z - jnp.log(jnp.sum(jnp.exp(z), axis=-1, keepdims=True))

    return pl.pallas_call(
        body,
        grid=(N // bn,),
        in_specs=[pl.BlockSpec((bn, H2), lambda i: (i, 0)),
                  pl.BlockSpec((bn, H2), lambda i: (i, 0)),
                  pl.BlockSpec((H, C), lambda i: (0, 0)),
                  pl.BlockSpec((1, C), lambda i: (0, 0))],
        out_specs=pl.BlockSpec((bn, C), lambda i: (i, 0)),
        out_shape=jax.ShapeDtypeStruct((N, C), jnp.float32),
    )


def kernel(x, edge_index, edge_weight, lin0_W, lin0_b, lin1_W, lin1_b, conv_W):
    N, F = x.shape
    H = lin0_W.shape[1]
    C = lin1_W.shape[1]
    E = edge_weight.shape[0]
    bn = 1000
    assert N % bn == 0

    eps = E // _NS
    ch = eps // _K
    src = edge_index[0].reshape(_NS, ch, _K)
    dst = edge_index[1].reshape(_NS, ch, _K)
    ew = edge_weight.reshape(_NS, eps)

    np_pad = _NS * 40 * ((N + _NS * 40 - 1) // (_NS * 40))
    sc_prop = _make_sc_prop(N, np_pad, E, H)
    lin0 = _make_lin0(N, F, H, bn)
    final = _make_final(N, H, C, bn)

    xlo, xhi = lin0(x, lin0_W, lin0_b.reshape(1, H))
    hlo, hhi = xlo, xhi
    for l in range(_NUM_LAYERS):
        beta = float(math.log(_THETA / (l + 1) + 1.0))
        plo, phi = sc_prop(hlo, hhi, src, dst, ew)
        hlo, hhi = _make_layer(N, np_pad, H, bn, beta)(
            plo, phi, xlo, xhi, conv_W[l])
    return final(hlo, hhi, lin1_W, lin1_b.reshape(1, C))


# speedup vs baseline: 1.9288x; 1.9288x over previous
import functools
import math

import jax
import jax.numpy as jnp
from jax import lax
from jax.experimental import pallas as pl
from jax.experimental.pallas import tpu as pltpu
from jax.experimental.pallas import tpu_sc as plsc

_NUM_LAYERS = 4
_ALPHA = 0.1
_THETA = 0.5

_NC = 2
_NS = 16
_L = 16

_K = 80
_NBUF = 5


def _make_sc_prop(N, Np, E, F):
    F2 = F // 2
    eps = E // _NS
    ch = eps // _K
    rps = Np // _NS
    zr = 40
    assert eps * _NS == E and ch * _K == eps and rps * _NS == Np
    assert rps % zr == 0 and rps % 8 == 0 and F2 % _L == 0
    assert ch % _NBUF == 0
    mesh = plsc.VectorSubcoreMesh(core_axis_name="c", subcore_axis_name="s")

    @functools.partial(
        pl.kernel,
        out_type=(jax.ShapeDtypeStruct((Np, F2), jnp.float32),
                  jax.ShapeDtypeStruct((Np, F2), jnp.float32)),
        mesh=mesh,
        compiler_params=pltpu.CompilerParams(
            needs_layout_passes=False, use_tc_tiling_on_sc=False),
        scratch_types=(
            [pltpu.VMEM((ch, _K), jnp.int32),
             pltpu.VMEM((ch, _K), jnp.int32),
             pltpu.VMEM((eps,), jnp.float32)]
            + [pltpu.VMEM((_K, F2), jnp.float32) for _ in range(_NBUF)]
            + [pltpu.VMEM((zr, F2), jnp.float32),
               pltpu.VMEM_SHARED((Np, F2), jnp.float32),
               pltpu.SemaphoreType.DMA((_NBUF,)),
               pltpu.SemaphoreType.DMA((_NBUF,))]
        ),
    )
    def sc_prop(hlo_hbm, hhi_hbm, src_hbm, dst_hbm, ew_hbm, plo_hbm, phi_hbm,
                src_v, dst_v, ew_v, *bufs):
        rows = list(bufs[:_NBUF])
        zero_v, acc_sh, gsem, ssem = bufs[_NBUF:]
        cid = lax.axis_index("c")
        sid = lax.axis_index("s")

        pltpu.sync_copy(src_hbm.at[sid], src_v)
        pltpu.sync_copy(dst_hbm.at[sid], dst_v)
        pltpu.sync_copy(ew_hbm.at[sid], ew_v)

        @pl.loop(0, zr)
        def _(r):
            for j in range(F2 // _L):
                zero_v[r, pl.ds(j * _L, _L)] = jnp.zeros((_L,), jnp.float32)

        @pl.loop(0, rps // zr)
        def _(i):
            pltpu.sync_copy(zero_v, acc_sh.at[pl.ds(sid * rps + i * zr, zr)])

        plsc.subcore_barrier()

        def scale_rows(rows_ref, t):
            @pl.loop(0, _K, unroll=4)
            def _(r):
                w = plsc.load_gather(
                    ew_v, [jnp.full((_L,), t * _K + r, jnp.int32)])
                for j in range(F2 // _L):
                    rows_ref[r, pl.ds(j * _L, _L)] = (
                        rows_ref[r, pl.ds(j * _L, _L)] * w)

        def main_pipeline(h_hbm):
            def g_fire(b, t):
                pltpu.async_copy(h_hbm.at[src_v.at[t]], rows[b], gsem.at[b])

            def g_wait(b, t):
                pltpu.make_async_copy(
                    h_hbm.at[src_v.at[t]], rows[b], gsem.at[b]).wait()

            def s_fire(b, t):
                pltpu.async_copy(rows[b], acc_sh.at[dst_v.at[t]],
                                 ssem.at[b], add=True)

            def s_wait(b, t):
                pltpu.make_async_copy(rows[b], acc_sh.at[dst_v.at[t]],
                                      ssem.at[b]).wait()

            for b in range(_NBUF):
                g_fire(b, b)

            @pl.loop(0, ch, step=_NBUF)
            def _(t):
                for b in range(_NBUF):
                    cur = t + b
                    g_wait(b, cur)
                    scale_rows(rows[b], cur)
                    s_fire(b, cur)
                    if b == 0:
                        @pl.when(t > 0)
                        def _():
                            s_wait(_NBUF - 1, t - 1)
                            g_fire(_NBUF - 1,
                                   jnp.minimum(t - 1 + _NBUF, ch - 1))
                    else:
                        s_wait(b - 1, cur - 1)
                        g_fire(b - 1, jnp.minimum(cur - 1 + _NBUF, ch - 1))

            s_wait(_NBUF - 1, ch - 1)
            for b in range(_NBUF - 1):
                g_wait(b, ch - 1)

        @pl.when(cid == 0)
        def _():
            main_pipeline(hlo_hbm)

        @pl.when(cid == 1)
        def _():
            main_pipeline(hhi_hbm)

        plsc.subcore_barrier()

        @pl.when(cid == 0)
        def _():
            pltpu.sync_copy(acc_sh.at[pl.ds(sid * rps, rps)],
                            plo_hbm.at[pl.ds(sid * rps, rps)])

        @pl.when(cid == 1)
        def _():
            pltpu.sync_copy(acc_sh.at[pl.ds(sid * rps, rps)],
                            phi_hbm.at[pl.ds(sid * rps, rps)])

    return sc_prop


def _make_lin0(N, F, H, bn):
    H2 = H // 2

    def body(x_ref, w_ref, b_ref, lo_ref, hi_ref):
        h = jnp.maximum(
            jnp.dot(x_ref[...], w_ref[...], preferred_element_type=jnp.float32)
            + b_ref[...], 0.0)
        lo_ref[...] = h[:, :H2]
        hi_ref[...] = h[:, H2:]

    return pl.pallas_call(
        body,
        grid=(N // bn,),
        in_specs=[pl.BlockSpec((bn, F), lambda i: (i, 0)),
                  pl.BlockSpec((F, H), lambda i: (0, 0)),
                  pl.BlockSpec((1, H), lambda i: (0, 0))],
        out_specs=[pl.BlockSpec((bn, H2), lambda i: (i, 0)),
                   pl.BlockSpec((bn, H2), lambda i: (i, 0))],
        out_shape=[jax.ShapeDtypeStruct((N, H2), jnp.float32),
                   jax.ShapeDtypeStruct((N, H2), jnp.float32)],
    )


def _make_layer(N, Np, H, bn, beta):
    H2 = H // 2

    def body(plo_ref, phi_ref, xlo_ref, xhi_ref, w_ref, lo_ref, hi_ref):
        agg = jnp.concatenate([plo_ref[...], phi_ref[...]], axis=1)
        x0 = jnp.concatenate([xlo_ref[...], xhi_ref[...]], axis=1)
        hh = (1.0 - _ALPHA) * agg + _ALPHA * x0
        out = (1.0 - beta) * hh + beta * jnp.dot(
            hh, w_ref[...], preferred_element_type=jnp.float32)
        out = jnp.maximum(out, 0.0)
        lo_ref[...] = out[:, :H2]
        hi_ref[...] = out[:, H2:]

    return pl.pallas_call(
        body,
        grid=(N // bn,),
        in_specs=[pl.BlockSpec((bn, H2), lambda i: (i, 0)),
                  pl.BlockSpec((bn, H2), lambda i: (i, 0)),
                  pl.BlockSpec((bn, H2), lambda i: (i, 0)),
                  pl.BlockSpec((bn, H2), lambda i: (i, 0)),
                  pl.BlockSpec((H, H), lambda i: (0, 0))],
        out_specs=[pl.BlockSpec((bn, H2), lambda i: (i, 0)),
                   pl.BlockSpec((bn, H2), lambda i: (i, 0))],
        out_shape=[jax.ShapeDtypeStruct((N, H2), jnp.float32),
                   jax.ShapeDtypeStruct((N, H2), jnp.float32)],
    )


def _make_final(N, H, C, bn):
    H2 = H // 2

    def body(lo_ref, hi_ref, w_ref, b_ref, o_ref):
        h = jnp.concatenate([lo_ref[...], hi_ref[...]], axis=1)
        logits = jnp.dot(h, w_ref[...],
                         preferred_element_type=jnp.float32) + b_ref[...]
        z = logits - jnp.max(logits, axis=-1, keepdims=True)
        o_ref[...] = z - jnp.log(jnp.sum(jnp.exp(z), axis=-1, keepdims=True))

    return pl.pallas_call(
        body,
        grid=(N // bn,),
        in_specs=[pl.BlockSpec((bn, H2), lambda i: (i, 0)),
                  pl.BlockSpec((bn, H2), lambda i: (i, 0)),
                  pl.BlockSpec((H, C), lambda i: (0, 0)),
                  pl.BlockSpec((1, C), lambda i: (0, 0))],
        out_specs=pl.BlockSpec((bn, C), lambda i: (i, 0)),
        out_shape=jax.ShapeDtypeStruct((N, C), jnp.float32),
    )


def kernel(x, edge_index, edge_weight, lin0_W, lin0_b, lin1_W, lin1_b, conv_W):
    N, F = x.shape
    H = lin0_W.shape[1]
    C = lin1_W.shape[1]
    E = edge_weight.shape[0]
    bn = 1000
    assert N % bn == 0

    eps = E // _NS
    ch = eps // _K
    src = edge_index[0].reshape(_NS, ch, _K)
    dst = edge_index[1].reshape(_NS, ch, _K)
    ew = edge_weight.reshape(_NS, eps)

    np_pad = _NS * 40 * ((N + _NS * 40 - 1) // (_NS * 40))
    sc_prop = _make_sc_prop(N, np_pad, E, H)
    lin0 = _make_lin0(N, F, H, bn)
    final = _make_final(N, H, C, bn)

    xlo, xhi = lin0(x, lin0_W, lin0_b.reshape(1, H))
    hlo, hhi = xlo, xhi
    for l in range(_NUM_LAYERS):
        beta = float(math.log(_THETA / (l + 1) + 1.0))
        plo, phi = sc_prop(hlo, hhi, src, dst, ew)
        hlo, hhi = _make_layer(N, np_pad, H, bn, beta)(
            plo, phi, xlo, xhi, conv_W[l])
    return final(hlo, hhi, lin1_W, lin1_b.reshape(1, C))
